# baseline (speedup 1.0000x reference)
import jax
import jax.numpy as jnp
from jax.experimental import pallas as pl
from jax.experimental.pallas import tpu as pltpu

_N_TOK = 32768
_T = 512
_E = 512
_C = 512
_H = 1024
_L = 128
_NB = _N_TOK // _T
_NCORES = 2
_STEPS = _NB // _NCORES


_NCHUNK = 1
_CH = _T // _NCHUNK


def _issue(tok_smem, emb_hbm, x_vmem, sems, step, buf):
    base = step * _T
    for c in range(_NCHUNK):
        for mi in range(_CH):
            r = c * _CH + mi
            tok = tok_smem[base + r]
            pltpu.make_async_copy(
                emb_hbm.at[tok], x_vmem.at[buf, r], sems.at[buf, c]).start()


def _body(tok_smem, emb_hbm, wc, bc, w1, b1, w2, b2, out_ref, x_vmem, sems):
    j = pl.program_id(0)
    par = jax.lax.rem(j, 2)

    @pl.when(j == 0)
    def _():
        _issue(tok_smem, emb_hbm, x_vmem, sems, 0, 0)

    def _step_work(buf, nxt):
        pltpu.make_async_copy(
            emb_hbm.at[pl.ds(0, _T)], x_vmem.at[buf], sems.at[buf, 0]).wait()
        nstep = jnp.minimum(j + 1, _NB - 1)
        base = nstep * _T
        x = x_vmem[buf]
        for mi in range(_T // 2):
            tok = tok_smem[base + mi]
            pltpu.make_async_copy(
                emb_hbm.at[tok], x_vmem.at[nxt, mi], sems.at[nxt, 0]).start()
        h = jnp.maximum(
            jnp.dot(x, wc[...], preferred_element_type=jnp.float32)
            + bc[...], 0.0)
        for mi in range(_T // 2, _T):
            tok = tok_smem[base + mi]
            pltpu.make_async_copy(
                emb_hbm.at[tok], x_vmem.at[nxt, mi], sems.at[nxt, 0]).start()
        z = jnp.maximum(
            jnp.dot(h, w1[...], preferred_element_type=jnp.float32)
            + b1[...], 0.0)
        out_ref[...] = (
            jnp.dot(z, w2[...], preferred_element_type=jnp.float32)
            + b2[...])

        @pl.when(j == _NB - 1)
        def _():
            pltpu.make_async_copy(
                emb_hbm.at[pl.ds(0, _T)], x_vmem.at[nxt],
                sems.at[nxt, 0]).wait()

    @pl.when(par == 0)
    def _():
        _step_work(0, 1)

    @pl.when(par == 1)
    def _():
        _step_work(1, 0)


def kernel(token_ids, emb_table, conv_w, conv_b, dec_w1, dec_b1, dec_w2,
           dec_b2, *, interpret=False):
    tokens = token_ids.reshape(-1).astype(jnp.int32)
    wc = conv_w.T
    bc = conv_b.reshape(1, _C)
    b1 = dec_b1.reshape(1, _H)
    b2 = dec_b2.reshape(1, _L)
    return pl.pallas_call(
        _body,
        out_shape=jax.ShapeDtypeStruct((_N_TOK, _L), jnp.float32),
        grid_spec=pltpu.PrefetchScalarGridSpec(
            num_scalar_prefetch=1,
            grid=(_NB,),
            in_specs=[
                pl.BlockSpec(memory_space=pl.ANY),
                pl.BlockSpec((_E, _C), lambda i, tok: (0, 0)),
                pl.BlockSpec((1, _C), lambda i, tok: (0, 0)),
                pl.BlockSpec((_C, _H), lambda i, tok: (0, 0)),
                pl.BlockSpec((1, _H), lambda i, tok: (0, 0)),
                pl.BlockSpec((_H, _L), lambda i, tok: (0, 0)),
                pl.BlockSpec((1, _L), lambda i, tok: (0, 0)),
            ],
            out_specs=pl.BlockSpec((_T, _L), lambda i, tok: (i, 0)),
            scratch_shapes=[
                pltpu.VMEM((2, _T, _E), jnp.float32),
                pltpu.SemaphoreType.DMA((2, _NCHUNK)),
            ],
        ),
        compiler_params=pltpu.CompilerParams(
            dimension_semantics=("arbitrary",),
        ),
        name="intent_slot_fused",
        interpret=interpret,
    )(tokens, emb_table, wc, bc, dec_w1, b1, dec_w2, b2)


# speedup vs baseline: 1.0373x; 1.0373x over previous
import jax
import jax.numpy as jnp
from jax.experimental import pallas as pl
from jax.experimental.pallas import tpu as pltpu

_N_TOK = 32768
_T = 512
_E = 512
_C = 512
_H = 1024
_L = 128
_NB = _N_TOK // _T
_NCORES = 2
_STEPS = _NB // _NCORES


_NCHUNK = 1
_CH = _T // _NCHUNK


def _issue(tok_smem, emb_hbm, x_vmem, sems, step, buf):
    base = step * _T
    for c in range(_NCHUNK):
        for mi in range(_CH):
            r = c * _CH + mi
            tok = tok_smem[base + r]
            pltpu.make_async_copy(
                emb_hbm.at[tok], x_vmem.at[buf, r],
                sems.at[buf, c]).start(priority=mi % 2)


def _body(tok_smem, emb_hbm, wc, bc, w1, b1, w2, b2, out_ref, x_vmem, sems):
    j = pl.program_id(0)
    par = jax.lax.rem(j, 2)

    @pl.when(j == 0)
    def _():
        _issue(tok_smem, emb_hbm, x_vmem, sems, 0, 0)

    def _step_work(buf, nxt):
        @pl.when(j < _NB - 1)
        def _():
            _issue(tok_smem, emb_hbm, x_vmem, sems, j + 1, nxt)
        pltpu.make_async_copy(
            emb_hbm.at[pl.ds(0, _T)], x_vmem.at[buf], sems.at[buf, 0]).wait()
        x = x_vmem[buf]
        h = jnp.maximum(
            jnp.dot(x, wc[...], preferred_element_type=jnp.float32)
            + bc[...], 0.0)
        z = jnp.maximum(
            jnp.dot(h, w1[...], preferred_element_type=jnp.float32)
            + b1[...], 0.0)
        out_ref[...] = (
            jnp.dot(z, w2[...], preferred_element_type=jnp.float32)
            + b2[...])

    @pl.when(par == 0)
    def _():
        _step_work(0, 1)

    @pl.when(par == 1)
    def _():
        _step_work(1, 0)


def kernel(token_ids, emb_table, conv_w, conv_b, dec_w1, dec_b1, dec_w2,
           dec_b2, *, interpret=False):
    tokens = token_ids.reshape(-1).astype(jnp.int32)
    wc = conv_w.T
    bc = conv_b.reshape(1, _C)
    b1 = dec_b1.reshape(1, _H)
    b2 = dec_b2.reshape(1, _L)
    return pl.pallas_call(
        _body,
        out_shape=jax.ShapeDtypeStruct((_N_TOK, _L), jnp.float32),
        grid_spec=pltpu.PrefetchScalarGridSpec(
            num_scalar_prefetch=1,
            grid=(_NB,),
            in_specs=[
                pl.BlockSpec(memory_space=pl.ANY),
                pl.BlockSpec((_E, _C), lambda i, tok: (0, 0)),
                pl.BlockSpec((1, _C), lambda i, tok: (0, 0)),
                pl.BlockSpec((_C, _H), lambda i, tok: (0, 0)),
                pl.BlockSpec((1, _H), lambda i, tok: (0, 0)),
                pl.BlockSpec((_H, _L), lambda i, tok: (0, 0)),
                pl.BlockSpec((1, _L), lambda i, tok: (0, 0)),
            ],
            out_specs=pl.BlockSpec((_T, _L), lambda i, tok: (i, 0)),
            scratch_shapes=[
                pltpu.VMEM((2, _T, _E), jnp.float32),
                pltpu.SemaphoreType.DMA((2, _NCHUNK)),
            ],
        ),
        compiler_params=pltpu.CompilerParams(
            dimension_semantics=("arbitrary",),
        ),
        name="intent_slot_fused",
        interpret=interpret,
    )(tokens, emb_table, wc, bc, dec_w1, b1, dec_w2, b2)


# speedup vs baseline: 1.0865x; 1.0474x over previous
import jax
import jax.numpy as jnp
from jax.experimental import pallas as pl
from jax.experimental.pallas import tpu as pltpu

_N_TOK = 32768
_T = 1024
_E = 512
_C = 512
_H = 1024
_L = 128
_NB = _N_TOK // _T
_NCORES = 2
_STEPS = _NB // _NCORES


_NCHUNK = 1
_CH = _T // _NCHUNK


def _issue(tok_smem, emb_hbm, x_vmem, sems, step, buf):
    base = step * _T
    for c in range(_NCHUNK):
        for mi in range(_CH):
            r = c * _CH + mi
            tok = tok_smem[base + r]
            pltpu.make_async_copy(
                emb_hbm.at[tok], x_vmem.at[buf, r],
                sems.at[buf, c]).start(priority=mi % 2)


def _body(tok_smem, emb_hbm, wc, bc, w1, b1, w2, b2, out_ref, x_vmem, sems):
    j = pl.program_id(0)
    par = jax.lax.rem(j, 2)

    @pl.when(j == 0)
    def _():
        _issue(tok_smem, emb_hbm, x_vmem, sems, 0, 0)

    def _step_work(buf, nxt):
        @pl.when(j < _NB - 1)
        def _():
            _issue(tok_smem, emb_hbm, x_vmem, sems, j + 1, nxt)
        pltpu.make_async_copy(
            emb_hbm.at[pl.ds(0, _T)], x_vmem.at[buf], sems.at[buf, 0]).wait()
        x = x_vmem[buf]
        h = jnp.maximum(
            jnp.dot(x, wc[...], preferred_element_type=jnp.float32)
            + bc[...], 0.0)
        z = jnp.maximum(
            jnp.dot(h, w1[...], preferred_element_type=jnp.float32)
            + b1[...], 0.0)
        out_ref[...] = (
            jnp.dot(z, w2[...], preferred_element_type=jnp.float32)
            + b2[...])

    @pl.when(par == 0)
    def _():
        _step_work(0, 1)

    @pl.when(par == 1)
    def _():
        _step_work(1, 0)


def kernel(token_ids, emb_table, conv_w, conv_b, dec_w1, dec_b1, dec_w2,
           dec_b2, *, interpret=False):
    tokens = token_ids.reshape(-1).astype(jnp.int32)
    wc = conv_w.T
    bc = conv_b.reshape(1, _C)
    b1 = dec_b1.reshape(1, _H)
    b2 = dec_b2.reshape(1, _L)
    return pl.pallas_call(
        _body,
        out_shape=jax.ShapeDtypeStruct((_N_TOK, _L), jnp.float32),
        grid_spec=pltpu.PrefetchScalarGridSpec(
            num_scalar_prefetch=1,
            grid=(_NB,),
            in_specs=[
                pl.BlockSpec(memory_space=pl.ANY),
                pl.BlockSpec((_E, _C), lambda i, tok: (0, 0)),
                pl.BlockSpec((1, _C), lambda i, tok: (0, 0)),
                pl.BlockSpec((_C, _H), lambda i, tok: (0, 0)),
                pl.BlockSpec((1, _H), lambda i, tok: (0, 0)),
                pl.BlockSpec((_H, _L), lambda i, tok: (0, 0)),
                pl.BlockSpec((1, _L), lambda i, tok: (0, 0)),
            ],
            out_specs=pl.BlockSpec((_T, _L), lambda i, tok: (i, 0)),
            scratch_shapes=[
                pltpu.VMEM((2, _T, _E), jnp.float32),
                pltpu.SemaphoreType.DMA((2, _NCHUNK)),
            ],
        ),
        compiler_params=pltpu.CompilerParams(
            dimension_semantics=("arbitrary",),
        ),
        name="intent_slot_fused",
        interpret=interpret,
    )(tokens, emb_table, wc, bc, dec_w1, b1, dec_w2, b2)


# speedup vs baseline: 1.1202x; 1.0311x over previous
import jax
import jax.numpy as jnp
from jax.experimental import pallas as pl
from jax.experimental.pallas import tpu as pltpu

_N_TOK = 32768
_T = 2048
_E = 512
_C = 512
_H = 1024
_L = 128
_NB = _N_TOK // _T
_NCORES = 2
_STEPS = _NB // _NCORES


_NCHUNK = 1
_CH = _T // _NCHUNK


def _issue(tok_smem, emb_hbm, x_vmem, sems, step, buf):
    base = step * _T
    for c in range(_NCHUNK):
        for mi in range(_CH):
            r = c * _CH + mi
            tok = tok_smem[base + r]
            pltpu.make_async_copy(
                emb_hbm.at[tok], x_vmem.at[buf, r],
                sems.at[buf, c]).start(priority=mi % 2)


def _body(tok_smem, emb_hbm, wc, bc, w1, b1, w2, b2, out_ref, x_vmem, sems):
    j = pl.program_id(0)
    par = jax.lax.rem(j, 2)

    @pl.when(j == 0)
    def _():
        _issue(tok_smem, emb_hbm, x_vmem, sems, 0, 0)

    def _step_work(buf, nxt):
        @pl.when(j < _NB - 1)
        def _():
            _issue(tok_smem, emb_hbm, x_vmem, sems, j + 1, nxt)
        pltpu.make_async_copy(
            emb_hbm.at[pl.ds(0, _T)], x_vmem.at[buf], sems.at[buf, 0]).wait()
        x = x_vmem[buf]
        h = jnp.maximum(
            jnp.dot(x, wc[...], preferred_element_type=jnp.float32)
            + bc[...], 0.0)
        z = jnp.maximum(
            jnp.dot(h, w1[...], preferred_element_type=jnp.float32)
            + b1[...], 0.0)
        out_ref[...] = (
            jnp.dot(z, w2[...], preferred_element_type=jnp.float32)
            + b2[...])

    @pl.when(par == 0)
    def _():
        _step_work(0, 1)

    @pl.when(par == 1)
    def _():
        _step_work(1, 0)


def kernel(token_ids, emb_table, conv_w, conv_b, dec_w1, dec_b1, dec_w2,
           dec_b2, *, interpret=False):
    tokens = token_ids.reshape(-1).astype(jnp.int32)
    wc = conv_w.T
    bc = conv_b.reshape(1, _C)
    b1 = dec_b1.reshape(1, _H)
    b2 = dec_b2.reshape(1, _L)
    return pl.pallas_call(
        _body,
        out_shape=jax.ShapeDtypeStruct((_N_TOK, _L), jnp.float32),
        grid_spec=pltpu.PrefetchScalarGridSpec(
            num_scalar_prefetch=1,
            grid=(_NB,),
            in_specs=[
                pl.BlockSpec(memory_space=pl.ANY),
                pl.BlockSpec((_E, _C), lambda i, tok: (0, 0)),
                pl.BlockSpec((1, _C), lambda i, tok: (0, 0)),
                pl.BlockSpec((_C, _H), lambda i, tok: (0, 0)),
                pl.BlockSpec((1, _H), lambda i, tok: (0, 0)),
                pl.BlockSpec((_H, _L), lambda i, tok: (0, 0)),
                pl.BlockSpec((1, _L), lambda i, tok: (0, 0)),
            ],
            out_specs=pl.BlockSpec((_T, _L), lambda i, tok: (i, 0)),
            scratch_shapes=[
                pltpu.VMEM((2, _T, _E), jnp.float32),
                pltpu.SemaphoreType.DMA((2, _NCHUNK)),
            ],
        ),
        compiler_params=pltpu.CompilerParams(
            dimension_semantics=("arbitrary",),
        ),
        name="intent_slot_fused",
        interpret=interpret,
    )(tokens, emb_table, wc, bc, dec_w1, b1, dec_w2, b2)
